# Initial kernel scaffold; baseline (speedup 1.0000x reference)
import jax
import jax.numpy as jnp
from jax.experimental import pallas as pl


def kernel(x_seq, Wx, bx, Wh, Wq, bq, concepts, cos_temp, Wg, bg, Wb, bb):
    raise NotImplementedError("write your pallas kernel here")


# speedup vs baseline: 4.2581x; 4.2581x over previous
import jax
import jax.numpy as jnp
from jax.experimental import pallas as pl
from jax.experimental.pallas import tpu as pltpu

B, N, D, H, V, CD = 4, 2048, 2048, 256, 8, 64
C = 2 ** V
ROWS = B * N
TILE = 512
NTILES = ROWS // TILE
TPB = N // TILE


def _mm_kernel(x_ref, wxT_ref, bx_ref, xw_ref, xsum_ref):
    i = pl.program_id(0)
    x = x_ref[...]
    xw_ref[...] = (
        jnp.dot(x, wxT_ref[...], preferred_element_type=jnp.float32) + bx_ref[...]
    )
    part = jnp.sum(x, axis=0, keepdims=True)[None]

    @pl.when(i % TPB == 0)
    def _init():
        xsum_ref[...] = part

    @pl.when(i % TPB != 0)
    def _acc():
        xsum_ref[...] += part


def _silu(a):
    return a / (1.0 + jnp.exp(-a))


def _scan_kernel(xw_ref, whT_ref, wgT_ref, bg_ref, xsum_ref, wqT_ref, bq_ref,
                 con_ref, temp_ref, comb_ref, out_ref):
    whT = whT_ref[...]

    def body(t, hs):
        new = []
        for b in range(B):
            a = xw_ref[b, pl.ds(t, 1), :] + jnp.dot(
                hs[b], whT, preferred_element_type=jnp.float32)
            new.append(_silu(a))
        return tuple(new)

    h0 = jnp.zeros((1, H), jnp.float32)
    hs = jax.lax.fori_loop(0, N, body, (h0,) * B)
    h = jnp.concatenate(hs, axis=0)

    gate = jnp.dot(h, wgT_ref[...], preferred_element_type=jnp.float32) + bg_ref[...]
    m = jnp.max(gate, axis=-1, keepdims=True)
    e = jnp.exp(gate - m)
    probs = e / jnp.sum(e, axis=-1, keepdims=True)
    mask = jnp.dot(probs, comb_ref[...], preferred_element_type=jnp.float32)
    mask = 0.99 * mask + 0.01

    xs = xsum_ref[...] * (1.0 / N)
    q = jnp.dot(xs, wqT_ref[...], preferred_element_type=jnp.float32) + bq_ref[...]
    q = _silu(q)
    qn = q / jnp.maximum(
        jnp.sqrt(jnp.sum(q * q, axis=-1, keepdims=True)), 1e-12)
    con = con_ref[...]
    cn = con / jnp.maximum(
        jnp.sqrt(jnp.sum(con * con, axis=-1, keepdims=True)), 1e-12)
    temp = jnp.maximum(temp_ref[0, 0], 0.01)
    sim = jax.lax.dot_general(
        qn, cn, (((1,), (1,)), ((), ())),
        preferred_element_type=jnp.float32) / temp

    z = sim
    col = jax.lax.broadcasted_iota(jnp.int32, (B, V), 1)
    gt = jnp.zeros_like(z)
    s = jnp.zeros_like(z)
    for j in range(V):
        zj = z[:, j:j + 1]
        g = jnp.where((zj > z) | ((zj == z) & (j < col)), 1.0, 0.0)
        gt = gt + g
        s = s + zj * g
    k = gt + 1.0
    s = s + z
    support = jnp.where(1.0 + k * z > s, 1.0, 0.0)
    k_z = jnp.maximum(jnp.sum(support, axis=-1, keepdims=True), 1.0)
    tau = (jnp.sum(z * support, axis=-1, keepdims=True) - 1.0) / k_z
    scores = jnp.maximum(z - tau, 0.0)

    out_ref[...] = mask * scores


def kernel(x_seq, Wx, bx, Wh, Wq, bq, concepts, cos_temp, Wg, bg, Wb, bb):
    f32 = jnp.float32
    x2 = x_seq.reshape(ROWS, D)
    xw, xsum3 = pl.pallas_call(
        _mm_kernel,
        grid=(NTILES,),
        in_specs=[
            pl.BlockSpec((TILE, D), lambda i: (i, 0)),
            pl.BlockSpec((D, H), lambda i: (0, 0)),
            pl.BlockSpec((1, H), lambda i: (0, 0)),
        ],
        out_specs=[
            pl.BlockSpec((TILE, H), lambda i: (i, 0)),
            pl.BlockSpec((1, 1, D), lambda i: (i // TPB, 0, 0)),
        ],
        out_shape=[
            jax.ShapeDtypeStruct((ROWS, H), f32),
            jax.ShapeDtypeStruct((B, 1, D), f32),
        ],
        compiler_params=pltpu.CompilerParams(
            dimension_semantics=("arbitrary",)),
    )(x2, Wx.T, bx.reshape(1, H))

    combo = jnp.arange(C, dtype=jnp.int32)[:, None]
    bits = 2 ** jnp.arange(V - 1, -1, -1, dtype=jnp.int32)
    comb = ((combo & bits) > 0).astype(f32)

    out = pl.pallas_call(
        _scan_kernel,
        out_shape=jax.ShapeDtypeStruct((B, V), f32),
    )(
        xw.reshape(B, N, H),
        Wh.T,
        Wg.T,
        bg.reshape(1, C),
        xsum3.reshape(B, D),
        Wq.T,
        bq.reshape(1, CD),
        concepts,
        cos_temp.reshape(1, 1),
        comb,
    )
    return out


# speedup vs baseline: 4.2628x; 1.0011x over previous
import jax
import jax.numpy as jnp
from jax.experimental import pallas as pl
from jax.experimental.pallas import tpu as pltpu

B, N, D, H, V, CD = 4, 2048, 2048, 256, 8, 64
C = 2 ** V
ROWS = B * N
TILE = 512
NTILES = ROWS // TILE
TPB = N // TILE


def _mm_kernel(x_ref, wxT_ref, bx_ref, xw_ref, xsum_ref):
    i = pl.program_id(0)
    x = x_ref[...]
    xw_ref[...] = (
        jnp.dot(x, wxT_ref[...], preferred_element_type=jnp.float32) + bx_ref[...]
    )
    part = jnp.sum(x, axis=0, keepdims=True)[None]

    @pl.when(i % TPB == 0)
    def _init():
        xsum_ref[...] = part

    @pl.when(i % TPB != 0)
    def _acc():
        xsum_ref[...] += part


def _silu(a):
    return a / (1.0 + jnp.exp(-a))


def _scan_kernel(xw_ref, whT_ref, h_ref):
    whT = whT_ref[...]

    def body(t, hs):
        new = []
        for b in range(B):
            a = xw_ref[b, pl.ds(t, 1), :] + jnp.dot(
                hs[b], whT, preferred_element_type=jnp.float32)
            new.append(_silu(a))
        return tuple(new)

    h0 = jnp.zeros((1, H), jnp.float32)
    hs = jax.lax.fori_loop(0, N, body, (h0,) * B)
    h_ref[...] = jnp.concatenate(hs, axis=0)


def _epilogue_kernel(h_ref, wgT_ref, bg_ref, xsum_ref, wqT_ref, bq_ref,
                     con_ref, temp_ref, comb_ref, out_ref):
    h = h_ref[...]

    gate = jnp.dot(h, wgT_ref[...], preferred_element_type=jnp.float32) + bg_ref[...]
    m = jnp.max(gate, axis=-1, keepdims=True)
    e = jnp.exp(gate - m)
    probs = e / jnp.sum(e, axis=-1, keepdims=True)
    mask = jnp.dot(probs, comb_ref[...], preferred_element_type=jnp.float32)
    mask = 0.99 * mask + 0.01

    xs = xsum_ref[...] * (1.0 / N)
    q = jnp.dot(xs, wqT_ref[...], preferred_element_type=jnp.float32) + bq_ref[...]
    q = _silu(q)
    qn = q / jnp.maximum(
        jnp.sqrt(jnp.sum(q * q, axis=-1, keepdims=True)), 1e-12)
    con = con_ref[...]
    cn = con / jnp.maximum(
        jnp.sqrt(jnp.sum(con * con, axis=-1, keepdims=True)), 1e-12)
    temp = jnp.maximum(temp_ref[0, 0], 0.01)
    sim = jax.lax.dot_general(
        qn, cn, (((1,), (1,)), ((), ())),
        preferred_element_type=jnp.float32) / temp

    z = sim
    col = jax.lax.broadcasted_iota(jnp.int32, (B, V), 1)
    gt = jnp.zeros_like(z)
    s = jnp.zeros_like(z)
    for j in range(V):
        zj = z[:, j:j + 1]
        g = jnp.where((zj > z) | ((zj == z) & (j < col)), 1.0, 0.0)
        gt = gt + g
        s = s + zj * g
    k = gt + 1.0
    s = s + z
    support = jnp.where(1.0 + k * z > s, 1.0, 0.0)
    k_z = jnp.maximum(jnp.sum(support, axis=-1, keepdims=True), 1.0)
    tau = (jnp.sum(z * support, axis=-1, keepdims=True) - 1.0) / k_z
    scores = jnp.maximum(z - tau, 0.0)

    out_ref[...] = mask * scores


def kernel(x_seq, Wx, bx, Wh, Wq, bq, concepts, cos_temp, Wg, bg, Wb, bb):
    f32 = jnp.float32
    x2 = x_seq.reshape(ROWS, D)
    xw, xsum3 = pl.pallas_call(
        _mm_kernel,
        grid=(NTILES,),
        in_specs=[
            pl.BlockSpec((TILE, D), lambda i: (i, 0)),
            pl.BlockSpec((D, H), lambda i: (0, 0)),
            pl.BlockSpec((1, H), lambda i: (0, 0)),
        ],
        out_specs=[
            pl.BlockSpec((TILE, H), lambda i: (i, 0)),
            pl.BlockSpec((1, 1, D), lambda i: (i // TPB, 0, 0)),
        ],
        out_shape=[
            jax.ShapeDtypeStruct((ROWS, H), f32),
            jax.ShapeDtypeStruct((B, 1, D), f32),
        ],
        compiler_params=pltpu.CompilerParams(
            dimension_semantics=("arbitrary",)),
    )(x2, Wx.T, bx.reshape(1, H))

    combo = jnp.arange(C, dtype=jnp.int32)[:, None]
    bits = 2 ** jnp.arange(V - 1, -1, -1, dtype=jnp.int32)
    comb = ((combo & bits) > 0).astype(f32)

    h = pl.pallas_call(
        _scan_kernel,
        out_shape=jax.ShapeDtypeStruct((B, H), f32),
    )(xw.reshape(B, N, H), Wh.T)

    out = pl.pallas_call(
        _epilogue_kernel,
        out_shape=jax.ShapeDtypeStruct((B, V), f32),
    )(
        h,
        Wg.T,
        bg.reshape(1, C),
        xsum3.reshape(B, D),
        Wq.T,
        bq.reshape(1, CD),
        concepts,
        cos_temp.reshape(1, 1),
        comb,
    )
    return out


# speedup vs baseline: 5.8228x; 1.3660x over previous
import jax
import jax.numpy as jnp
from jax.experimental import pallas as pl
from jax.experimental.pallas import tpu as pltpu

B, N, D, H, V, CD = 4, 2048, 2048, 256, 8, 64
C = 2 ** V
ROWS = B * N
TILE = 512
NTILES = ROWS // TILE
TPB = N // TILE


def _mm_kernel(x_ref, wxT_ref, bx_ref, xw_ref, xsum_ref):
    i = pl.program_id(0)
    x = x_ref[...]
    xw_ref[...] = (
        jnp.dot(x, wxT_ref[...], preferred_element_type=jnp.float32) + bx_ref[...]
    )
    part = jnp.sum(x, axis=0, keepdims=True)[None]

    @pl.when(i % TPB == 0)
    def _init():
        xsum_ref[...] = part

    @pl.when(i % TPB != 0)
    def _acc():
        xsum_ref[...] += part


def _silu(a):
    return a / (1.0 + jnp.exp(-a))


def _scan_kernel(xw_ref, whT_ref, h_ref):
    whT = whT_ref[...]

    def body(t, hs):
        new = []
        for b in range(B):
            a = xw_ref[b, pl.ds(t, 1), :] + jnp.dot(
                hs[b], whT, preferred_element_type=jnp.float32)
            new.append(_silu(a))
        return tuple(new)

    h0 = jnp.zeros((1, H), jnp.float32)
    hs = jax.lax.fori_loop(0, N, body, (h0,) * B, unroll=8)
    h_ref[...] = jnp.concatenate(hs, axis=0)


def _epilogue_kernel(h_ref, wgT_ref, bg_ref, xsum_ref, wqT_ref, bq_ref,
                     con_ref, temp_ref, comb_ref, out_ref):
    h = h_ref[...]

    gate = jnp.dot(h, wgT_ref[...], preferred_element_type=jnp.float32) + bg_ref[...]
    m = jnp.max(gate, axis=-1, keepdims=True)
    e = jnp.exp(gate - m)
    probs = e / jnp.sum(e, axis=-1, keepdims=True)
    mask = jnp.dot(probs, comb_ref[...], preferred_element_type=jnp.float32)
    mask = 0.99 * mask + 0.01

    xs = xsum_ref[...] * (1.0 / N)
    q = jnp.dot(xs, wqT_ref[...], preferred_element_type=jnp.float32) + bq_ref[...]
    q = _silu(q)
    qn = q / jnp.maximum(
        jnp.sqrt(jnp.sum(q * q, axis=-1, keepdims=True)), 1e-12)
    con = con_ref[...]
    cn = con / jnp.maximum(
        jnp.sqrt(jnp.sum(con * con, axis=-1, keepdims=True)), 1e-12)
    temp = jnp.maximum(temp_ref[0, 0], 0.01)
    sim = jax.lax.dot_general(
        qn, cn, (((1,), (1,)), ((), ())),
        preferred_element_type=jnp.float32) / temp

    z = sim
    col = jax.lax.broadcasted_iota(jnp.int32, (B, V), 1)
    gt = jnp.zeros_like(z)
    s = jnp.zeros_like(z)
    for j in range(V):
        zj = z[:, j:j + 1]
        g = jnp.where((zj > z) | ((zj == z) & (j < col)), 1.0, 0.0)
        gt = gt + g
        s = s + zj * g
    k = gt + 1.0
    s = s + z
    support = jnp.where(1.0 + k * z > s, 1.0, 0.0)
    k_z = jnp.maximum(jnp.sum(support, axis=-1, keepdims=True), 1.0)
    tau = (jnp.sum(z * support, axis=-1, keepdims=True) - 1.0) / k_z
    scores = jnp.maximum(z - tau, 0.0)

    out_ref[...] = mask * scores


def kernel(x_seq, Wx, bx, Wh, Wq, bq, concepts, cos_temp, Wg, bg, Wb, bb):
    f32 = jnp.float32
    x2 = x_seq.reshape(ROWS, D)
    xw, xsum3 = pl.pallas_call(
        _mm_kernel,
        grid=(NTILES,),
        in_specs=[
            pl.BlockSpec((TILE, D), lambda i: (i, 0)),
            pl.BlockSpec((D, H), lambda i: (0, 0)),
            pl.BlockSpec((1, H), lambda i: (0, 0)),
        ],
        out_specs=[
            pl.BlockSpec((TILE, H), lambda i: (i, 0)),
            pl.BlockSpec((1, 1, D), lambda i: (i // TPB, 0, 0)),
        ],
        out_shape=[
            jax.ShapeDtypeStruct((ROWS, H), f32),
            jax.ShapeDtypeStruct((B, 1, D), f32),
        ],
        compiler_params=pltpu.CompilerParams(
            dimension_semantics=("arbitrary",)),
    )(x2, Wx.T, bx.reshape(1, H))

    combo = jnp.arange(C, dtype=jnp.int32)[:, None]
    bits = 2 ** jnp.arange(V - 1, -1, -1, dtype=jnp.int32)
    comb = ((combo & bits) > 0).astype(f32)

    h = pl.pallas_call(
        _scan_kernel,
        out_shape=jax.ShapeDtypeStruct((B, H), f32),
    )(xw.reshape(B, N, H), Wh.T)

    out = pl.pallas_call(
        _epilogue_kernel,
        out_shape=jax.ShapeDtypeStruct((B, V), f32),
    )(
        h,
        Wg.T,
        bg.reshape(1, C),
        xsum3.reshape(B, D),
        Wq.T,
        bq.reshape(1, CD),
        concepts,
        cos_temp.reshape(1, 1),
        comb,
    )
    return out


# speedup vs baseline: 6.1243x; 1.0518x over previous
import jax
import jax.numpy as jnp
from jax.experimental import pallas as pl
from jax.experimental.pallas import tpu as pltpu

B, N, D, H, V, CD = 4, 2048, 2048, 256, 8, 64
C = 2 ** V
TILE = 512
SCAN = 128
JMAX = N // TILE


def _silu(a):
    return a * (0.5 + 0.5 * jnp.tanh(0.5 * a))


def _fused_kernel(x_ref, wxT_ref, bx_ref, whT_ref, wgT_ref, bg_ref,
                  wqT_ref, bq_ref, con_ref, temp_ref, comb_ref,
                  out_ref, xw_s, xsum_s, h_s):
    j = pl.program_id(0)
    b = pl.program_id(1)

    @pl.when(j < JMAX)
    def _matmul_tile():
        x = x_ref[...].reshape(TILE, D)
        xw = jnp.dot(x, wxT_ref[...],
                     preferred_element_type=jnp.float32) + bx_ref[...]
        xw_s[b, pl.ds(j * TILE, TILE), :] = xw
        part = jnp.sum(x, axis=0, keepdims=True)

        @pl.when(j == 0)
        def _init():
            xsum_s[b] = part

        @pl.when(j != 0)
        def _acc():
            xsum_s[b] += part

    c = JMAX * j + b - JMAX

    @pl.when(c == 0)
    def _h_init():
        h_s[...] = jnp.zeros((B, H), jnp.float32)

    @pl.when((c >= 0) & (c < N // SCAN))
    def _scan_chunk():
        whT = whT_ref[...]
        t0 = c * SCAN

        def body(i, h):
            xt = xw_s[:, pl.ds(t0 + i, 1), :].reshape(B, H)
            mm = jnp.dot(h.astype(jnp.bfloat16), whT,
                         preferred_element_type=jnp.float32)
            return _silu(xt + mm)

        h_s[...] = jax.lax.fori_loop(0, SCAN, body, h_s[...], unroll=8)

    @pl.when((j == JMAX) & (b == B - 1))
    def _epilogue():
        h = h_s[...]
        gate = jnp.dot(h, wgT_ref[...],
                       preferred_element_type=jnp.float32) + bg_ref[...]
        m = jnp.max(gate, axis=-1, keepdims=True)
        e = jnp.exp(gate - m)
        probs = e / jnp.sum(e, axis=-1, keepdims=True)
        mask = jnp.dot(probs, comb_ref[...],
                       preferred_element_type=jnp.float32)
        mask = 0.99 * mask + 0.01

        xs = xsum_s[...].reshape(B, D) * (1.0 / N)
        q = jnp.dot(xs, wqT_ref[...],
                    preferred_element_type=jnp.float32) + bq_ref[...]
        q = _silu(q)
        qn = q / jnp.maximum(
            jnp.sqrt(jnp.sum(q * q, axis=-1, keepdims=True)), 1e-12)
        con = con_ref[...]
        cn = con / jnp.maximum(
            jnp.sqrt(jnp.sum(con * con, axis=-1, keepdims=True)), 1e-12)
        temp = jnp.maximum(temp_ref[0, 0], 0.01)
        sim = jax.lax.dot_general(
            qn, cn, (((1,), (1,)), ((), ())),
            preferred_element_type=jnp.float32) / temp

        z = sim
        col = jax.lax.broadcasted_iota(jnp.int32, (B, V), 1)
        gt = jnp.zeros_like(z)
        s = jnp.zeros_like(z)
        for jj in range(V):
            zj = z[:, jj:jj + 1]
            g = jnp.where((zj > z) | ((zj == z) & (jj < col)), 1.0, 0.0)
            gt = gt + g
            s = s + zj * g
        k = gt + 1.0
        s = s + z
        support = jnp.where(1.0 + k * z > s, 1.0, 0.0)
        k_z = jnp.maximum(jnp.sum(support, axis=-1, keepdims=True), 1.0)
        tau = (jnp.sum(z * support, axis=-1, keepdims=True) - 1.0) / k_z
        scores = jnp.maximum(z - tau, 0.0)

        out_ref[...] = mask * scores


def kernel(x_seq, Wx, bx, Wh, Wq, bq, concepts, cos_temp, Wg, bg, Wb, bb):
    f32 = jnp.float32
    combo = jnp.arange(C, dtype=jnp.int32)[:, None]
    bits = 2 ** jnp.arange(V - 1, -1, -1, dtype=jnp.int32)
    comb = ((combo & bits) > 0).astype(f32)

    full = lambda shape: pl.BlockSpec(shape, lambda j, b: (0,) * len(shape))
    out = pl.pallas_call(
        _fused_kernel,
        grid=(JMAX + 1, B),
        in_specs=[
            pl.BlockSpec((1, TILE, D),
                         lambda j, b: (b, jnp.minimum(j, JMAX - 1), 0)),
            full((D, H)),
            full((1, H)),
            full((H, H)),
            full((H, C)),
            full((1, C)),
            full((D, CD)),
            full((1, CD)),
            full((V, CD)),
            full((1, 1)),
            full((C, V)),
        ],
        out_specs=pl.BlockSpec((B, V), lambda j, b: (0, 0)),
        out_shape=jax.ShapeDtypeStruct((B, V), f32),
        scratch_shapes=[
            pltpu.VMEM((B, N, H), f32),
            pltpu.VMEM((B, 1, D), f32),
            pltpu.VMEM((B, H), f32),
        ],
        compiler_params=pltpu.CompilerParams(
            dimension_semantics=("arbitrary", "arbitrary")),
    )(
        x_seq,
        Wx.T,
        bx.reshape(1, H),
        Wh.T.astype(jnp.bfloat16),
        Wg.T,
        bg.reshape(1, C),
        Wq.T,
        bq.reshape(1, CD),
        concepts,
        cos_temp.reshape(1, 1),
        comb,
    )
    return out
